# baseline (speedup 1.0000x reference)
import functools

import jax
import jax.numpy as jnp
from jax import lax
from jax.experimental import pallas as pl
from jax.experimental.pallas import tpu as pltpu
from jax.experimental.pallas import tpu_sc as plsc

VOCAB = 256
VPAD = VOCAB + 8
EMBED = 64
B = 4096
L = 200

NC = 2
NS = 16
NW = NC * NS

BW = B // NW
NL = 2
NCHUNK = L // NL
NBUF = 2

_MESH = plsc.VectorSubcoreMesh(core_axis_name="c", subcore_axis_name="s")


@functools.partial(
    pl.kernel,
    out_type=jax.ShapeDtypeStruct((L, EMBED // 8, NW, 8, 128), jnp.float32),
    mesh=_MESH,
    compiler_params=pltpu.CompilerParams(
        use_tc_tiling_on_sc=False, needs_layout_passes=False
    ),
    scratch_types=[
        pltpu.VMEM((EMBED * VPAD,), jnp.float32),
        pltpu.VMEM((BW,), jnp.int32),
        pltpu.VMEM((2, NL, BW), jnp.int32),
        pltpu.VMEM((NBUF, NL, EMBED // 8, 1, 8, 128), jnp.float32),
        pltpu.SemaphoreType.DMA,
        pltpu.SemaphoreType.DMA,
        pltpu.SemaphoreType.DMA,
        pltpu.SemaphoreType.DMA,
        pltpu.SemaphoreType.DMA,
    ],
)
def _emb_kernel(
    ttab_hbm, idxt_hbm, len_hbm, out_hbm,
    tab_v, len_v, idx_raw, stage_v,
    sem_idx0, sem_idx1, sem_out0, sem_out1, sem_tab,
):
    sem_idx = (sem_idx0, sem_idx1)
    sem_out = (sem_out0, sem_out1)

    sid = lax.axis_index("s")
    wid = lax.axis_index("c") * NS + sid
    b0 = wid * BW

    tab_cp = pltpu.async_copy(ttab_hbm, tab_v, sem_tab)
    pltpu.sync_copy(len_hbm.at[pl.ds(b0, BW)], len_v)

    def idx_pair(gg, bi):
        return idxt_hbm.at[pl.ds(gg * NL, NL), pl.ds(b0, BW)], idx_raw.at[bi]

    def out_pair(gg, bo):
        return (
            stage_v.at[bo],
            out_hbm.at[pl.ds(gg * NL, NL), pl.ds(0, EMBED // 8), pl.ds(wid, 1)],
        )

    def start_idx(gg, bi):
        src, dst = idx_pair(gg, bi)
        pltpu.async_copy(src, dst, sem_idx[bi])

    def compute(gg, bi, bo):
        src, dst = idx_pair(gg, bi)
        pltpu.make_async_copy(src, dst, sem_idx[bi]).wait()
        for lofs in range(NL):
            lpos = gg * NL + lofs
            for cg in range(BW // 16):
                lenv = len_v[pl.ds(cg * 16, 16)]
                rawv = idx_raw[bi, lofs, pl.ds(cg * 16, 16)]
                iv = jnp.where(lenv > lpos, rawv, VOCAB)

                def store_e(e, val):
                    stage_v[
                        bo, lofs, e // 8, 0, e % 8, pl.ds(cg * 16, 16)
                    ] = val

                DIST = 10
                vals = {}
                for e in range(EMBED + DIST):
                    if e < EMBED:
                        vals[e] = plsc.load_gather(tab_v, [iv + (e * VPAD)])
                    if e >= DIST:
                        store_e(e - DIST, vals.pop(e - DIST))

    start_idx(0, 0)
    start_idx(1, 1)
    tab_cp.wait()

    @pl.loop(0, NCHUNK, step=2)
    def _(g):
        for db in (0, 1):
            gg = g + db
            bi = db
            bo = db

            @pl.when(gg >= NBUF)
            def _(gg=gg, bo=bo):
                src, dst = out_pair(gg - NBUF, bo)
                pltpu.make_async_copy(src, dst, sem_out[bo]).wait()

            compute(gg, bi, bo)

            @pl.when(gg + 2 < NCHUNK)
            def _(gg=gg, bi=bi):
                start_idx(gg + 2, bi)

            src, dst = out_pair(gg, bo)
            pltpu.async_copy(src, dst, sem_out[bo])

    for gg in (NCHUNK - 2, NCHUNK - 1):
        src, dst = out_pair(gg, gg % 2)
        pltpu.make_async_copy(src, dst, sem_out[gg % 2]).wait()


def kernel(vectorized_seqs, seq_lengths, weight):
    idx_t = vectorized_seqs.T
    ttab = jnp.concatenate(
        [weight.T, jnp.zeros((EMBED, VPAD - VOCAB), jnp.float32)], axis=1
    ).reshape(EMBED * VPAD)
    out5 = _emb_kernel(ttab, idx_t, seq_lengths)
    return out5.transpose(2, 4, 0, 1, 3).reshape(B, L, EMBED)


# speedup vs baseline: 1.0336x; 1.0336x over previous
import functools

import jax
import jax.numpy as jnp
from jax import lax
from jax.experimental import pallas as pl
from jax.experimental.pallas import tpu as pltpu
from jax.experimental.pallas import tpu_sc as plsc

VOCAB = 256
VPAD = VOCAB + 8
EMBED = 64
B = 4096
L = 200

NC = 2
NS = 16
NW = NC * NS

BW = B // NW
NL = 2
NCHUNK = L // NL
NBUF = 2

_MESH = plsc.VectorSubcoreMesh(core_axis_name="c", subcore_axis_name="s")


@functools.partial(
    pl.kernel,
    out_type=jax.ShapeDtypeStruct((L, EMBED // 8, NW, 8, 128), jnp.float32),
    mesh=_MESH,
    compiler_params=pltpu.CompilerParams(
        use_tc_tiling_on_sc=False, needs_layout_passes=False
    ),
    scratch_types=[
        pltpu.VMEM((EMBED * VPAD,), jnp.float32),
        pltpu.VMEM((BW,), jnp.int32),
        pltpu.VMEM((2, NL, BW), jnp.int32),
        pltpu.VMEM((NBUF, NL, EMBED // 8, 1, 8, 128), jnp.float32),
        pltpu.SemaphoreType.DMA,
        pltpu.SemaphoreType.DMA,
        pltpu.SemaphoreType.DMA,
        pltpu.SemaphoreType.DMA,
        pltpu.SemaphoreType.DMA,
    ],
)
def _emb_kernel(
    ttab_hbm, idxt_hbm, len_hbm, out_hbm,
    tab_v, len_v, idx_raw, stage_v,
    sem_idx0, sem_idx1, sem_out0, sem_out1, sem_tab,
):
    sem_idx = (sem_idx0, sem_idx1)
    sem_out = (sem_out0, sem_out1)

    sid = lax.axis_index("s")
    wid = lax.axis_index("c") * NS + sid
    b0 = wid * BW

    tab_cp = pltpu.async_copy(ttab_hbm, tab_v, sem_tab)
    pltpu.sync_copy(len_hbm.at[pl.ds(b0, BW)], len_v)

    def idx_pair(gg, bi):
        return idxt_hbm.at[pl.ds(gg * NL, NL), pl.ds(b0, BW)], idx_raw.at[bi]

    def out_pair(gg, bo):
        return (
            stage_v.at[bo],
            out_hbm.at[pl.ds(gg * NL, NL), pl.ds(0, EMBED // 8), pl.ds(wid, 1)],
        )

    def start_idx(gg, bi):
        src, dst = idx_pair(gg, bi)
        pltpu.async_copy(src, dst, sem_idx[bi])

    def compute(gg, bi, bo):
        src, dst = idx_pair(gg, bi)
        pltpu.make_async_copy(src, dst, sem_idx[bi]).wait()
        for lofs in range(NL):
            lpos = gg * NL + lofs
            for cg in range(BW // 16):
                lenv = len_v[pl.ds(cg * 16, 16)]
                rawv = idx_raw[bi, lofs, pl.ds(cg * 16, 16)]
                iv = jnp.where(lenv > lpos, rawv, VOCAB)

                def store_e(e, val):
                    stage_v[
                        bo, lofs, e // 8, 0, e % 8, pl.ds(cg * 16, 16)
                    ] = val

                DIST = 7
                vals = {}
                for e in range(EMBED + DIST):
                    if e < EMBED:
                        vals[e] = plsc.load_gather(tab_v, [iv + (e * VPAD)])
                    if e >= DIST:
                        store_e(e - DIST, vals.pop(e - DIST))

    start_idx(0, 0)
    start_idx(1, 1)
    tab_cp.wait()

    @pl.loop(0, NCHUNK, step=2)
    def _(g):
        for db in (0, 1):
            gg = g + db
            bi = db
            bo = db

            @pl.when(gg >= NBUF)
            def _(gg=gg, bo=bo):
                src, dst = out_pair(gg - NBUF, bo)
                pltpu.make_async_copy(src, dst, sem_out[bo]).wait()

            compute(gg, bi, bo)

            @pl.when(gg + 2 < NCHUNK)
            def _(gg=gg, bi=bi):
                start_idx(gg + 2, bi)

            src, dst = out_pair(gg, bo)
            pltpu.async_copy(src, dst, sem_out[bo])

    for gg in (NCHUNK - 2, NCHUNK - 1):
        src, dst = out_pair(gg, gg % 2)
        pltpu.make_async_copy(src, dst, sem_out[gg % 2]).wait()


def kernel(vectorized_seqs, seq_lengths, weight):
    idx_t = vectorized_seqs.T
    ttab = jnp.concatenate(
        [weight.T, jnp.zeros((EMBED, VPAD - VOCAB), jnp.float32)], axis=1
    ).reshape(EMBED * VPAD)
    out5 = _emb_kernel(ttab, idx_t, seq_lengths)
    return out5.transpose(2, 4, 0, 1, 3).reshape(B, L, EMBED)


# speedup vs baseline: 1.7751x; 1.7174x over previous
import functools

import jax
import jax.numpy as jnp
from jax import lax
from jax.experimental import pallas as pl
from jax.experimental.pallas import tpu as pltpu
from jax.experimental.pallas import tpu_sc as plsc

VOCAB = 256
VPAD = VOCAB + 8
EMBED = 64
B = 4096
L = 200

NC = 2
NS = 16
NW = NC * NS

BW = B // NW
NL = 1
NCHUNK = L // NL
NBUF = 2

_MESH = plsc.VectorSubcoreMesh(core_axis_name="c", subcore_axis_name="s")


@functools.partial(
    pl.kernel,
    out_type=jax.ShapeDtypeStruct((L, EMBED // 8, NW, 8, 128), jnp.float32),
    mesh=_MESH,
    compiler_params=pltpu.CompilerParams(
        use_tc_tiling_on_sc=False, needs_layout_passes=False
    ),
    scratch_types=[
        pltpu.VMEM((EMBED * VPAD,), jnp.float32),
        pltpu.VMEM((BW,), jnp.int32),
        pltpu.VMEM((2, NL, BW), jnp.int32),
        pltpu.VMEM((NBUF, NL, EMBED // 8, 1, 8, 128), jnp.float32),
        pltpu.SemaphoreType.DMA,
        pltpu.SemaphoreType.DMA,
        pltpu.SemaphoreType.DMA,
        pltpu.SemaphoreType.DMA,
        pltpu.SemaphoreType.DMA,
    ],
)
def _emb_kernel(
    ttab_hbm, idxt_hbm, len_hbm, out_hbm,
    tab_v, len_v, idx_raw, stage_v,
    sem_idx0, sem_idx1, sem_out0, sem_out1, sem_tab,
):
    sem_idx = (sem_idx0, sem_idx1)
    sem_out = (sem_out0, sem_out1)

    sid = lax.axis_index("s")
    wid = lax.axis_index("c") * NS + sid
    b0 = wid * BW

    tab_cp = pltpu.async_copy(ttab_hbm, tab_v, sem_tab)
    pltpu.sync_copy(len_hbm.at[pl.ds(b0, BW)], len_v)

    def idx_pair(gg, bi):
        return idxt_hbm.at[pl.ds(gg * NL, NL), pl.ds(b0, BW)], idx_raw.at[bi]

    def out_pair(gg, bo):
        return (
            stage_v.at[bo],
            out_hbm.at[pl.ds(gg * NL, NL), pl.ds(0, EMBED // 8), pl.ds(wid, 1)],
        )

    def start_idx(gg, bi):
        src, dst = idx_pair(gg, bi)
        pltpu.async_copy(src, dst, sem_idx[bi])

    def compute(gg, bi, bo):
        src, dst = idx_pair(gg, bi)
        pltpu.make_async_copy(src, dst, sem_idx[bi]).wait()
        for lofs in range(NL):
            lpos = gg * NL + lofs
            for cg in range(BW // 16):
                lenv = len_v[pl.ds(cg * 16, 16)]
                rawv = idx_raw[bi, lofs, pl.ds(cg * 16, 16)]
                iv = jnp.where(lenv > lpos, rawv, VOCAB)

                def store_e(e, val):
                    stage_v[
                        bo, lofs, e // 8, 0, e % 8, pl.ds(cg * 16, 16)
                    ] = val

                DIST = 6
                vals = {}
                for e in range(EMBED + DIST):
                    if e < EMBED:
                        vals[e] = plsc.load_gather(tab_v, [iv + (e * VPAD)])
                    if e >= DIST:
                        store_e(e - DIST, vals.pop(e - DIST))

    start_idx(0, 0)
    start_idx(1, 1)
    tab_cp.wait()

    @pl.loop(0, NCHUNK, step=2)
    def _(g):
        for db in (0, 1):
            gg = g + db
            bi = db
            bo = db

            @pl.when(gg >= NBUF)
            def _(gg=gg, bo=bo):
                src, dst = out_pair(gg - NBUF, bo)
                pltpu.make_async_copy(src, dst, sem_out[bo]).wait()

            compute(gg, bi, bo)

            @pl.when(gg + 2 < NCHUNK)
            def _(gg=gg, bi=bi):
                start_idx(gg + 2, bi)

            src, dst = out_pair(gg, bo)
            pltpu.async_copy(src, dst, sem_out[bo])

    for gg in (NCHUNK - 2, NCHUNK - 1):
        src, dst = out_pair(gg, gg % 2)
        pltpu.make_async_copy(src, dst, sem_out[gg % 2]).wait()


def kernel(vectorized_seqs, seq_lengths, weight):
    idx_t = vectorized_seqs.T
    ttab = jnp.concatenate(
        [weight.T, jnp.zeros((EMBED, VPAD - VOCAB), jnp.float32)], axis=1
    ).reshape(EMBED * VPAD)
    out5 = _emb_kernel(ttab, idx_t, seq_lengths)
    return out5.transpose(2, 4, 0, 1, 3).reshape(B, L, EMBED)


# speedup vs baseline: 1.7901x; 1.0084x over previous
import functools

import jax
import jax.numpy as jnp
from jax import lax
from jax.experimental import pallas as pl
from jax.experimental.pallas import tpu as pltpu
from jax.experimental.pallas import tpu_sc as plsc

VOCAB = 256
VPAD = VOCAB + 8
EMBED = 64
B = 4096
L = 200

NC = 2
NS = 16
NW = NC * NS

BW = B // NW
NL = 1
NCHUNK = L // NL
NBUF = 2

_MESH = plsc.VectorSubcoreMesh(core_axis_name="c", subcore_axis_name="s")


@functools.partial(
    pl.kernel,
    out_type=jax.ShapeDtypeStruct((L, EMBED // 8, NW, 8, 128), jnp.float32),
    mesh=_MESH,
    compiler_params=pltpu.CompilerParams(
        use_tc_tiling_on_sc=False, needs_layout_passes=False
    ),
    scratch_types=[
        pltpu.VMEM((EMBED * VPAD,), jnp.float32),
        pltpu.VMEM_SHARED((EMBED * VPAD,), jnp.float32),
        pltpu.VMEM((BW,), jnp.int32),
        pltpu.VMEM((2, NL, BW), jnp.int32),
        pltpu.VMEM((NBUF, NL, EMBED // 8, 1, 8, 128), jnp.float32),
        pltpu.SemaphoreType.DMA,
        pltpu.SemaphoreType.DMA,
        pltpu.SemaphoreType.DMA,
        pltpu.SemaphoreType.DMA,
        pltpu.SemaphoreType.DMA,
    ],
)
def _emb_kernel(
    ttab_hbm, idxt_hbm, len_hbm, out_hbm,
    tab_v, tab_sh, len_v, idx_raw, stage_v,
    sem_idx0, sem_idx1, sem_out0, sem_out1, sem_tab,
):
    sem_idx = (sem_idx0, sem_idx1)
    sem_out = (sem_out0, sem_out1)

    sid = lax.axis_index("s")
    wid = lax.axis_index("c") * NS + sid
    b0 = wid * BW

    @pl.when(sid == 0)
    def _():
        pltpu.sync_copy(ttab_hbm, tab_sh)

    pltpu.sync_copy(len_hbm.at[pl.ds(b0, BW)], len_v)
    plsc.subcore_barrier()
    tab_cp = pltpu.async_copy(tab_sh, tab_v, sem_tab)

    def idx_pair(gg, bi):
        return idxt_hbm.at[pl.ds(gg * NL, NL), pl.ds(b0, BW)], idx_raw.at[bi]

    def out_pair(gg, bo):
        return (
            stage_v.at[bo],
            out_hbm.at[pl.ds(gg * NL, NL), pl.ds(0, EMBED // 8), pl.ds(wid, 1)],
        )

    def start_idx(gg, bi):
        src, dst = idx_pair(gg, bi)
        pltpu.async_copy(src, dst, sem_idx[bi])

    def compute(gg, bi, bo):
        src, dst = idx_pair(gg, bi)
        pltpu.make_async_copy(src, dst, sem_idx[bi]).wait()
        for lofs in range(NL):
            lpos = gg * NL + lofs
            for cg in range(BW // 16):
                lenv = len_v[pl.ds(cg * 16, 16)]
                rawv = idx_raw[bi, lofs, pl.ds(cg * 16, 16)]
                iv = jnp.where(lenv > lpos, rawv, VOCAB)

                def store_e(e, val):
                    stage_v[
                        bo, lofs, e // 8, 0, e % 8, pl.ds(cg * 16, 16)
                    ] = val

                DIST = 6
                vals = {}
                for e in range(EMBED + DIST):
                    if e < EMBED:
                        vals[e] = plsc.load_gather(tab_v, [iv + (e * VPAD)])
                    if e >= DIST:
                        store_e(e - DIST, vals.pop(e - DIST))

    start_idx(0, 0)
    start_idx(1, 1)
    tab_cp.wait()

    @pl.loop(0, NCHUNK, step=2)
    def _(g):
        for db in (0, 1):
            gg = g + db
            bi = db
            bo = db

            @pl.when(gg >= NBUF)
            def _(gg=gg, bo=bo):
                src, dst = out_pair(gg - NBUF, bo)
                pltpu.make_async_copy(src, dst, sem_out[bo]).wait()

            compute(gg, bi, bo)

            @pl.when(gg + 2 < NCHUNK)
            def _(gg=gg, bi=bi):
                start_idx(gg + 2, bi)

            src, dst = out_pair(gg, bo)
            pltpu.async_copy(src, dst, sem_out[bo])

    for gg in (NCHUNK - 2, NCHUNK - 1):
        src, dst = out_pair(gg, gg % 2)
        pltpu.make_async_copy(src, dst, sem_out[gg % 2]).wait()


def kernel(vectorized_seqs, seq_lengths, weight):
    idx_t = vectorized_seqs.T
    ttab = jnp.concatenate(
        [weight.T, jnp.zeros((EMBED, VPAD - VOCAB), jnp.float32)], axis=1
    ).reshape(EMBED * VPAD)
    out5 = _emb_kernel(ttab, idx_t, seq_lengths)
    return out5.transpose(2, 4, 0, 1, 3).reshape(B, L, EMBED)
